# Initial kernel scaffold; baseline (speedup 1.0000x reference)
import jax
import jax.numpy as jnp
from jax.experimental import pallas as pl


def kernel(Z, random_planes):
    raise NotImplementedError("write your pallas kernel here")


# speedup vs baseline: 6.7205x; 6.7205x over previous
import functools

import jax
import jax.numpy as jnp
from jax.experimental import pallas as pl
from jax.experimental.pallas import tpu as pltpu

N = 4096
D = 1024
BANDS = 16
ROWS = 8
SIM_THRESH = 0.5
TI = 512


def _prologue_kernel(z_ref, planes_t_ref, zn_ref, keys_ref):
    z = z_ref[...]
    nrm2 = jnp.sum(z * z, axis=1, keepdims=True)
    zn_ref[...] = (z * jax.lax.rsqrt(nrm2)).astype(jnp.bfloat16)
    s = jnp.dot(z, planes_t_ref[...], preferred_element_type=jnp.float32,
                precision=jax.lax.Precision.HIGHEST)
    bits = (s >= 0.0).astype(jnp.bfloat16)
    k_idx = jax.lax.broadcasted_iota(jnp.int32, (BANDS * ROWS, BANDS), 0)
    b_idx = jax.lax.broadcasted_iota(jnp.int32, (BANDS * ROWS, BANDS), 1)
    w = jnp.where(k_idx // ROWS == b_idx,
                  jnp.left_shift(1, k_idx % ROWS), 0).astype(jnp.bfloat16)
    keys_ref[...] = jnp.dot(bits, w, preferred_element_type=jnp.float32)


def _main_kernel(zi_ref, znt_ref, ki_ref, kb_ref, out_ref):
    g = jnp.dot(zi_ref[...], znt_ref[...], preferred_element_type=jnp.float32)
    i0 = pl.program_id(0) * TI
    ri = jax.lax.broadcasted_iota(jnp.int32, (TI, N), 0) + i0
    ci = jax.lax.broadcasted_iota(jnp.int32, (TI, N), 1)
    keep = (g >= SIM_THRESH) & (ri != ci)
    masked = jnp.where(keep, g, 0.0)
    out_ref[...] = masked
    hit = jnp.max(masked) > 0.0

    @pl.when(hit)
    def _():
        ki = ki_ref[...]
        kb = kb_ref[...]
        cnt = jnp.zeros((TI, N), jnp.float32)
        for b in range(BANDS):
            cnt = cnt + (ki[:, b:b + 1] == kb[b:b + 1, :]).astype(jnp.float32)
        out_ref[...] = masked * cnt


@functools.partial(jax.jit, static_argnames=())
def kernel(Z, random_planes):
    planes_t = random_planes.T
    zn, keys = pl.pallas_call(
        _prologue_kernel,
        out_shape=(
            jax.ShapeDtypeStruct((N, D), jnp.bfloat16),
            jax.ShapeDtypeStruct((N, BANDS), jnp.float32),
        ),
        compiler_params=pltpu.CompilerParams(vmem_limit_bytes=120 * 2**20),
    )(Z, planes_t)
    znt = zn.T
    keys_b = keys.T
    out = pl.pallas_call(
        _main_kernel,
        grid=(N // TI,),
        in_specs=[
            pl.BlockSpec((TI, D), lambda i: (i, 0)),
            pl.BlockSpec((D, N), lambda i: (0, 0)),
            pl.BlockSpec((TI, BANDS), lambda i: (i, 0)),
            pl.BlockSpec((BANDS, N), lambda i: (0, 0)),
        ],
        out_specs=pl.BlockSpec((TI, N), lambda i: (i, 0)),
        out_shape=jax.ShapeDtypeStruct((N, N), jnp.float32),
        compiler_params=pltpu.CompilerParams(vmem_limit_bytes=120 * 2**20),
    )(zn, znt, keys, keys_b)
    return out


# speedup vs baseline: 9.8089x; 1.4596x over previous
import functools

import jax
import jax.numpy as jnp
from jax.experimental import pallas as pl
from jax.experimental.pallas import tpu as pltpu

N = 4096
D = 1024
BANDS = 16
ROWS = 8
SIM_THRESH = 0.5
TI = 512


def _prologue_kernel(z_ref, planes_t_ref, zn_ref, keys_ref):
    z = z_ref[...]
    nrm2 = jnp.sum(z * z, axis=1, keepdims=True)
    zn_ref[...] = (z * jax.lax.rsqrt(nrm2)).astype(jnp.bfloat16)
    s = jnp.dot(z, planes_t_ref[...], preferred_element_type=jnp.float32)
    bits = (s >= 0.0).astype(jnp.bfloat16)
    k_idx = jax.lax.broadcasted_iota(jnp.int32, (BANDS * ROWS, BANDS), 0)
    b_idx = jax.lax.broadcasted_iota(jnp.int32, (BANDS * ROWS, BANDS), 1)
    w = jnp.where(k_idx // ROWS == b_idx,
                  jnp.left_shift(1, k_idx % ROWS), 0).astype(jnp.bfloat16)
    keys_ref[...] = jnp.dot(bits, w, preferred_element_type=jnp.float32)


def _main_kernel(zi_ref, zn_ref, ki_ref, kb_ref, out_ref):
    g = jax.lax.dot_general(zi_ref[...], zn_ref[...],
                            dimension_numbers=(((1,), (1,)), ((), ())),
                            preferred_element_type=jnp.float32)
    i0 = pl.program_id(0) * TI
    cmr = (jax.lax.broadcasted_iota(jnp.int32, (TI, N), 1)
           - jax.lax.broadcasted_iota(jnp.int32, (TI, N), 0))
    masked = jnp.where(cmr == i0, 0.0, jnp.where(g >= SIM_THRESH, g, 0.0))
    out_ref[...] = masked
    hit = jnp.max(masked) > 0.0

    @pl.when(hit)
    def _():
        ki = ki_ref[...]
        kb = kb_ref[...]
        cnt = jnp.zeros((TI, N), jnp.float32)
        for b in range(BANDS):
            cnt = cnt + (ki[:, b:b + 1] == kb[b:b + 1, :]).astype(jnp.float32)
        out_ref[...] = masked * cnt


@functools.partial(jax.jit, static_argnames=())
def kernel(Z, random_planes):
    planes_t = random_planes.T
    zn, keys = pl.pallas_call(
        _prologue_kernel,
        out_shape=(
            jax.ShapeDtypeStruct((N, D), jnp.bfloat16),
            jax.ShapeDtypeStruct((N, BANDS), jnp.float32),
        ),
        compiler_params=pltpu.CompilerParams(vmem_limit_bytes=120 * 2**20),
    )(Z, planes_t)
    keys_b = keys.T
    out = pl.pallas_call(
        _main_kernel,
        grid=(N // TI,),
        in_specs=[
            pl.BlockSpec((TI, D), lambda i: (i, 0)),
            pl.BlockSpec((N, D), lambda i: (0, 0)),
            pl.BlockSpec((TI, BANDS), lambda i: (i, 0)),
            pl.BlockSpec((BANDS, N), lambda i: (0, 0)),
        ],
        out_specs=pl.BlockSpec((TI, N), lambda i: (i, 0)),
        out_shape=jax.ShapeDtypeStruct((N, N), jnp.float32),
        compiler_params=pltpu.CompilerParams(vmem_limit_bytes=120 * 2**20),
    )(zn, zn, keys, keys_b)
    return out


# speedup vs baseline: 12.9809x; 1.3234x over previous
import functools

import jax
import jax.numpy as jnp
from jax.experimental import pallas as pl
from jax.experimental.pallas import tpu as pltpu

N = 4096
D = 1024
BANDS = 16
ROWS = 8
SIM_THRESH = 0.5
TI = 512


def _prologue_kernel(z_ref, planes_t_ref, zn_ref, keys_ref):
    z = z_ref[...]
    nrm2 = jnp.sum(z * z, axis=1, keepdims=True)
    zn_ref[...] = (z * (16.0 * jax.lax.rsqrt(nrm2))).astype(jnp.float8_e4m3fn)
    s = jnp.dot(z, planes_t_ref[...], preferred_element_type=jnp.float32)
    bits = (s >= 0.0).astype(jnp.bfloat16)
    k_idx = jax.lax.broadcasted_iota(jnp.int32, (BANDS * ROWS, BANDS), 0)
    b_idx = jax.lax.broadcasted_iota(jnp.int32, (BANDS * ROWS, BANDS), 1)
    w = jnp.where(k_idx // ROWS == b_idx,
                  jnp.left_shift(1, k_idx % ROWS), 0).astype(jnp.bfloat16)
    keys_ref[...] = jnp.dot(bits, w, preferred_element_type=jnp.float32)


def _main_kernel(zi_ref, zn_ref, ki_ref, kb_ref, out_ref):
    g = jax.lax.dot_general(zi_ref[...], zn_ref[...],
                            dimension_numbers=(((1,), (1,)), ((), ())),
                            preferred_element_type=jnp.float32)
    i0 = pl.program_id(0) * TI
    cmr = (jax.lax.broadcasted_iota(jnp.int32, (TI, N), 1)
           - jax.lax.broadcasted_iota(jnp.int32, (TI, N), 0))
    masked = jnp.where(cmr == i0, 0.0,
                       jnp.where(g >= 256.0 * SIM_THRESH, g * (1.0 / 256.0),
                                 0.0))
    out_ref[...] = masked
    hit = jnp.max(masked) > 0.0

    @pl.when(hit)
    def _():
        ki = ki_ref[...]
        kb = kb_ref[...]
        cnt = jnp.zeros((TI, N), jnp.float32)
        for b in range(BANDS):
            cnt = cnt + (ki[:, b:b + 1] == kb[b:b + 1, :]).astype(jnp.float32)
        out_ref[...] = masked * cnt


@functools.partial(jax.jit, static_argnames=())
def kernel(Z, random_planes):
    planes_t = random_planes.T
    zn, keys = pl.pallas_call(
        _prologue_kernel,
        out_shape=(
            jax.ShapeDtypeStruct((N, D), jnp.float8_e4m3fn),
            jax.ShapeDtypeStruct((N, BANDS), jnp.float32),
        ),
        compiler_params=pltpu.CompilerParams(vmem_limit_bytes=120 * 2**20),
    )(Z, planes_t)
    keys_b = keys.T
    out = pl.pallas_call(
        _main_kernel,
        grid=(N // TI,),
        in_specs=[
            pl.BlockSpec((TI, D), lambda i: (i, 0)),
            pl.BlockSpec((N, D), lambda i: (0, 0)),
            pl.BlockSpec((TI, BANDS), lambda i: (i, 0)),
            pl.BlockSpec((BANDS, N), lambda i: (0, 0)),
        ],
        out_specs=pl.BlockSpec((TI, N), lambda i: (i, 0)),
        out_shape=jax.ShapeDtypeStruct((N, N), jnp.float32),
        compiler_params=pltpu.CompilerParams(vmem_limit_bytes=120 * 2**20),
    )(zn, zn, keys, keys_b)
    return out


# speedup vs baseline: 14.4271x; 1.1114x over previous
import functools

import jax
import jax.numpy as jnp
from jax.experimental import pallas as pl
from jax.experimental.pallas import tpu as pltpu

N = 4096
D = 1024
BANDS = 16
ROWS = 8
SIM_THRESH = 0.5
TP = 512
TI = 512
CHUNK = 1024


def _prologue_kernel(z_ref, planes_t_ref, zn_ref, keys_ref):
    z = z_ref[...]
    nrm2 = jnp.sum(z * z, axis=1, keepdims=True)
    zn_ref[...] = (z * jax.lax.rsqrt(nrm2)).astype(jnp.float8_e5m2)
    s = jnp.dot(z, planes_t_ref[...], preferred_element_type=jnp.float32)
    bits = (s >= 0.0).astype(jnp.bfloat16)
    k_idx = jax.lax.broadcasted_iota(jnp.int32, (BANDS * ROWS, BANDS), 0)
    b_idx = jax.lax.broadcasted_iota(jnp.int32, (BANDS * ROWS, BANDS), 1)
    w = jnp.where(k_idx // ROWS == b_idx,
                  jnp.left_shift(1, k_idx % ROWS), 0).astype(jnp.bfloat16)
    keys_ref[...] = jnp.dot(bits, w, preferred_element_type=jnp.float32)


def _main_kernel(zi_ref, zn_ref, ki_ref, kb_ref, out_ref):
    i0 = pl.program_id(0) * TI
    zi = zi_ref[...]
    m = jnp.float32(0.0)
    for c in range(N // CHUNK):
        g = jax.lax.dot_general(zi, zn_ref[c * CHUNK:(c + 1) * CHUNK, :],
                                dimension_numbers=(((1,), (1,)), ((), ())),
                                preferred_element_type=jnp.float32)
        cmr = (jax.lax.broadcasted_iota(jnp.int32, (TI, CHUNK), 1) + c * CHUNK
               - jax.lax.broadcasted_iota(jnp.int32, (TI, CHUNK), 0))
        masked = jnp.where(cmr == i0, 0.0,
                           jnp.where(g >= SIM_THRESH, g, 0.0))
        out_ref[:, c * CHUNK:(c + 1) * CHUNK] = masked
        m = jnp.maximum(m, jnp.max(masked))

    @pl.when(m > 0.0)
    def _():
        ki = ki_ref[...]
        kb = kb_ref[...]
        cnt = jnp.zeros((TI, N), jnp.float32)
        for b in range(BANDS):
            cnt = cnt + (ki[:, b:b + 1] == kb[b:b + 1, :]).astype(jnp.float32)
        out_ref[...] = out_ref[...] * cnt


@functools.partial(jax.jit, static_argnames=())
def kernel(Z, random_planes):
    planes_t = random_planes.T
    zn, keys = pl.pallas_call(
        _prologue_kernel,
        grid=(N // TP,),
        in_specs=[
            pl.BlockSpec((TP, D), lambda i: (i, 0)),
            pl.BlockSpec((D, BANDS * ROWS), lambda i: (0, 0)),
        ],
        out_specs=(
            pl.BlockSpec((TP, D), lambda i: (i, 0)),
            pl.BlockSpec((TP, BANDS), lambda i: (i, 0)),
        ),
        out_shape=(
            jax.ShapeDtypeStruct((N, D), jnp.float8_e5m2),
            jax.ShapeDtypeStruct((N, BANDS), jnp.float32),
        ),
        compiler_params=pltpu.CompilerParams(vmem_limit_bytes=100 * 2**20),
    )(Z, planes_t)
    keys_b = keys.T
    out = pl.pallas_call(
        _main_kernel,
        grid=(N // TI,),
        in_specs=[
            pl.BlockSpec((TI, D), lambda i: (i, 0)),
            pl.BlockSpec((N, D), lambda i: (0, 0)),
            pl.BlockSpec((TI, BANDS), lambda i: (i, 0)),
            pl.BlockSpec((BANDS, N), lambda i: (0, 0)),
        ],
        out_specs=pl.BlockSpec((TI, N), lambda i: (i, 0)),
        out_shape=jax.ShapeDtypeStruct((N, N), jnp.float32),
        compiler_params=pltpu.CompilerParams(vmem_limit_bytes=100 * 2**20),
    )(zn, zn, keys, keys_b)
    return out


# speedup vs baseline: 16.8368x; 1.1670x over previous
import functools

import jax
import jax.numpy as jnp
from jax.experimental import pallas as pl
from jax.experimental.pallas import tpu as pltpu

N = 4096
D = 1024
BANDS = 16
ROWS = 8
SIM_THRESH = 0.5
TI = 512
CHUNK = 1024


def _fused_kernel(z_ref, planes_t_ref, out_ref, zn_s, keys_s, kb_s):
    pid = pl.program_id(0)

    @pl.when(pid == 0)
    def _prologue():
        z = z_ref[...]
        nrm2 = jnp.sum(z * z, axis=1, keepdims=True)
        zn_s[...] = (z * jax.lax.rsqrt(nrm2)).astype(jnp.float8_e5m2)
        s = jnp.dot(z, planes_t_ref[...], preferred_element_type=jnp.float32)
        bits = (s >= 0.0).astype(jnp.bfloat16)
        k_idx = jax.lax.broadcasted_iota(jnp.int32, (BANDS * ROWS, BANDS), 0)
        b_idx = jax.lax.broadcasted_iota(jnp.int32, (BANDS * ROWS, BANDS), 1)
        w = jnp.where(k_idx // ROWS == b_idx,
                      jnp.left_shift(1, k_idx % ROWS), 0).astype(jnp.bfloat16)
        keys = jnp.dot(bits, w, preferred_element_type=jnp.float32)
        keys_s[...] = keys
        kb_s[...] = keys.T

    @pl.when(pid != 0)
    def _slab():
        i0 = (pid - 1) * TI
        zi = zn_s[pl.ds(i0, TI), :]
        m = jnp.float32(0.0)
        for c in range(N // CHUNK):
            g = jax.lax.dot_general(zi, zn_s[c * CHUNK:(c + 1) * CHUNK, :],
                                    dimension_numbers=(((1,), (1,)), ((), ())),
                                    preferred_element_type=jnp.float32)
            cmr = (jax.lax.broadcasted_iota(jnp.int32, (TI, CHUNK), 1)
                   + c * CHUNK
                   - jax.lax.broadcasted_iota(jnp.int32, (TI, CHUNK), 0))
            masked = jnp.where(cmr == i0, 0.0,
                               jnp.where(g >= SIM_THRESH, g, 0.0))
            out_ref[:, c * CHUNK:(c + 1) * CHUNK] = masked
            m = jnp.maximum(m, jnp.max(masked))

        @pl.when(m > 0.0)
        def _counts():
            ki = keys_s[pl.ds(i0, TI), :]
            kb = kb_s[...]
            cnt = jnp.zeros((TI, N), jnp.float32)
            for b in range(BANDS):
                cnt = cnt + (ki[:, b:b + 1] == kb[b:b + 1, :]).astype(
                    jnp.float32)
            out_ref[...] = out_ref[...] * cnt


@functools.partial(jax.jit, static_argnames=())
def kernel(Z, random_planes):
    planes_t = random_planes.T
    out = pl.pallas_call(
        _fused_kernel,
        grid=(1 + N // TI,),
        in_specs=[
            pl.BlockSpec((N, D), lambda i: (0, 0)),
            pl.BlockSpec((D, BANDS * ROWS), lambda i: (0, 0)),
        ],
        out_specs=pl.BlockSpec((TI, N),
                               lambda i: (jnp.maximum(i - 1, 0), 0)),
        out_shape=jax.ShapeDtypeStruct((N, N), jnp.float32),
        scratch_shapes=[
            pltpu.VMEM((N, D), jnp.float8_e5m2),
            pltpu.VMEM((N, BANDS), jnp.float32),
            pltpu.VMEM((BANDS, N), jnp.float32),
        ],
        compiler_params=pltpu.CompilerParams(vmem_limit_bytes=100 * 2**20),
    )(Z, planes_t)
    return out
